# baseline (device time: 277034 ns/iter reference)
import jax
import jax.numpy as jnp
from jax import lax
from jax.experimental import pallas as pl
from jax.experimental.pallas import tpu as pltpu

B = 32
H = 16
D = 128
BS = 32
X_SIZE = 2

PP = 8

NEG_INF = -1e30

_CompilerParams = getattr(pltpu, "CompilerParams", None) or getattr(
    pltpu, "TPUCompilerParams"
)


def _flash_body(q_ref, k_ref, v_ref, w_ref, acc_ref, m_ref, l_ref):
    p = pl.program_id(0)

    @pl.when(p == 0)
    def _():
        acc_ref[...] = jnp.zeros_like(acc_ref)
        m_ref[...] = jnp.full_like(m_ref, NEG_INF)
        l_ref[...] = jnp.zeros_like(l_ref)

    w = w_ref[...]
    for h in range(H):
        sl = slice(h * D, (h + 1) * D)
        q = q_ref[:, sl]
        k = k_ref[:, sl]
        v = v_ref[:, sl]

        s = lax.dot_general(
            q, k, (((1,), (1,)), ((), ())),
            preferred_element_type=jnp.float32,
        ) * (D ** -0.5)

        m_old = m_ref[:, h : h + 1]
        m_new = jnp.maximum(m_old, jnp.max(s, axis=1, keepdims=True))
        alpha = jnp.exp(m_old - m_new)
        e = w * jnp.exp(s - m_new)
        pv = lax.dot_general(
            e, v, (((1,), (0,)), ((), ())),
            preferred_element_type=jnp.float32,
        )

        m_ref[:, h : h + 1] = m_new
        l_ref[:, h : h + 1] = (
            alpha * l_ref[:, h : h + 1] + jnp.sum(e, axis=1, keepdims=True)
        )
        acc_ref[:, sl] = alpha * acc_ref[:, sl] + pv


def _merge_body(
    acc_ref, m_ref, l_ref, out_ref, racc, rm, rl, send_sems, recv_sems
):
    my_x = lax.axis_index("x")
    my_y = lax.axis_index("y")
    my_z = lax.axis_index("z")
    partner = (1 - my_x, my_y, my_z)

    bar = pltpu.get_barrier_semaphore()
    pl.semaphore_signal(
        bar, inc=1, device_id=partner, device_id_type=pl.DeviceIdType.MESH
    )
    pl.semaphore_wait(bar, 1)

    copies = []
    for i, (src, dst) in enumerate(
        [(acc_ref, racc), (m_ref, rm), (l_ref, rl)]
    ):
        copies.append(
            pltpu.make_async_remote_copy(
                src_ref=src,
                dst_ref=dst,
                send_sem=send_sems.at[i],
                recv_sem=recv_sems.at[i],
                device_id=partner,
                device_id_type=pl.DeviceIdType.MESH,
            )
        )
    for c in copies:
        c.start()
    for c in copies:
        c.wait()

    m_a = m_ref[...]
    m_b = rm[...]
    m_star = jnp.maximum(m_a, m_b)
    c_a = jnp.exp(m_a - m_star)
    c_b = jnp.exp(m_b - m_star)
    l_star = c_a * l_ref[...] + c_b * rl[...]
    num = c_a[:, :, None] * acc_ref[...] + c_b[:, :, None] * racc[...]
    out_ref[...] = num / l_star[:, :, None]


def kernel(Q, K, V, bt, lens):
    np_local = K.shape[0]
    n_slots = bt.shape[1]
    my_x = lax.axis_index("x")

    base = my_x * np_local
    slot = jnp.arange(n_slots)
    valid = slot[None, :] < lens[:, None]
    local_id = jnp.where(valid, bt, -1) - base
    w = jnp.sum(
        (local_id[:, :, None] == jnp.arange(np_local)[None, None, :]).astype(
            jnp.float32
        ),
        axis=1,
    )
    w_tok = jnp.repeat(w, BS, axis=1)

    q2 = Q.reshape(B, H * D)
    k2 = K.reshape(np_local * BS, H * D)
    v2 = V.reshape(np_local * BS, H * D)

    n_steps = np_local // PP
    acc, m, l = pl.pallas_call(
        _flash_body,
        grid=(n_steps,),
        in_specs=[
            pl.BlockSpec((B, H * D), lambda p: (0, 0)),
            pl.BlockSpec((PP * BS, H * D), lambda p: (p, 0)),
            pl.BlockSpec((PP * BS, H * D), lambda p: (p, 0)),
            pl.BlockSpec((B, PP * BS), lambda p: (0, p)),
        ],
        out_specs=[
            pl.BlockSpec((B, H * D), lambda p: (0, 0)),
            pl.BlockSpec((B, H), lambda p: (0, 0)),
            pl.BlockSpec((B, H), lambda p: (0, 0)),
        ],
        out_shape=[
            jax.ShapeDtypeStruct((B, H * D), jnp.float32),
            jax.ShapeDtypeStruct((B, H), jnp.float32),
            jax.ShapeDtypeStruct((B, H), jnp.float32),
        ],
        compiler_params=_CompilerParams(
            dimension_semantics=("arbitrary",)
        ),
    )(q2, k2, v2, w_tok)
    acc = acc.reshape(B, H, D)

    out = pl.pallas_call(
        _merge_body,
        out_shape=jax.ShapeDtypeStruct((B, H, D), jnp.float32),
        in_specs=[
            pl.BlockSpec(memory_space=pltpu.VMEM),
            pl.BlockSpec(memory_space=pltpu.VMEM),
            pl.BlockSpec(memory_space=pltpu.VMEM),
        ],
        out_specs=pl.BlockSpec(memory_space=pltpu.VMEM),
        scratch_shapes=[
            pltpu.VMEM((B, H, D), jnp.float32),
            pltpu.VMEM((B, H), jnp.float32),
            pltpu.VMEM((B, H), jnp.float32),
            pltpu.SemaphoreType.DMA((3,)),
            pltpu.SemaphoreType.DMA((3,)),
        ],
        compiler_params=_CompilerParams(collective_id=0),
    )(acc, m, l)

    return out[:, None, :, :]


# device time: 54754 ns/iter; 5.0596x vs baseline; 5.0596x over previous
import jax
import jax.numpy as jnp
from jax import lax
from jax.experimental import pallas as pl
from jax.experimental.pallas import tpu as pltpu

B = 32
H = 16
D = 128
BS = 32
X_SIZE = 2

_CompilerParams = getattr(pltpu, "CompilerParams", None) or getattr(
    pltpu, "TPUCompilerParams"
)


def _flash_body(
    qt_ref, w_ref, k_hbm, v_hbm, acc_ref, l_ref, k_buf, v_buf, k_sems, v_sems
):
    np_local, bs = k_hbm.shape[0], k_hbm.shape[1]
    t = np_local * bs

    def copy(hbm, buf, sems, h, slot):
        return pltpu.make_async_copy(
            hbm.at[:, :, h, :], buf.at[slot], sems.at[slot]
        )

    copy(k_hbm, k_buf, k_sems, 0, 0).start()
    copy(v_hbm, v_buf, v_sems, 0, 0).start()
    w = w_ref[...]
    for h in range(H):
        slot = h % 2
        if h + 1 < H:
            copy(k_hbm, k_buf, k_sems, h + 1, 1 - slot).start()
            copy(v_hbm, v_buf, v_sems, h + 1, 1 - slot).start()
        copy(k_hbm, k_buf, k_sems, h, slot).wait()
        q = qt_ref[h]
        k = k_buf[slot].reshape(t, D)
        s = lax.dot_general(
            q, k, (((1,), (1,)), ((), ())),
            preferred_element_type=jnp.float32,
        )
        e = w * jnp.exp(s)
        l_ref[:, h : h + 1] = jnp.sum(e, axis=1, keepdims=True)
        copy(v_hbm, v_buf, v_sems, h, slot).wait()
        v = v_buf[slot].reshape(t, D)
        acc_ref[h] = lax.dot_general(
            e, v, (((1,), (0,)), ((), ())),
            preferred_element_type=jnp.float32,
        )


def _merge_body(acc_ref, l_ref, out_ref, racc, rl, send_sems, recv_sems):
    my_x = lax.axis_index("x")
    my_y = lax.axis_index("y")
    my_z = lax.axis_index("z")
    partner = (1 - my_x, my_y, my_z)

    bar = pltpu.get_barrier_semaphore()
    pl.semaphore_signal(
        bar, inc=1, device_id=partner, device_id_type=pl.DeviceIdType.MESH
    )
    pl.semaphore_wait(bar, 1)

    copies = []
    for i, (src, dst) in enumerate([(acc_ref, racc), (l_ref, rl)]):
        copies.append(
            pltpu.make_async_remote_copy(
                src_ref=src,
                dst_ref=dst,
                send_sem=send_sems.at[i],
                recv_sem=recv_sems.at[i],
                device_id=partner,
                device_id_type=pl.DeviceIdType.MESH,
            )
        )
    for c in copies:
        c.start()
    for c in copies:
        c.wait()

    l_star = l_ref[...] + rl[...]
    for h in range(H):
        out_ref[:, h, :] = (acc_ref[h] + racc[h]) / l_star[:, h : h + 1]


def kernel(Q, K, V, bt, lens):
    np_local = K.shape[0]
    n_slots = bt.shape[1]
    t_local = np_local * BS
    my_x = lax.axis_index("x")

    base = my_x * np_local
    slot = jnp.arange(n_slots)
    valid = slot[None, :] < lens[:, None]
    local_id = jnp.where(valid, bt, -1) - base
    w = jnp.sum(
        (local_id[:, :, None] == jnp.arange(np_local)[None, None, :]).astype(
            jnp.float32
        ),
        axis=1,
    )
    w_tok = jnp.repeat(w, BS, axis=1)

    qt = Q[:, 0].transpose(1, 0, 2) * (D ** -0.5)

    acc, l = pl.pallas_call(
        _flash_body,
        in_specs=[
            pl.BlockSpec(memory_space=pltpu.VMEM),
            pl.BlockSpec(memory_space=pltpu.VMEM),
            pl.BlockSpec(memory_space=pltpu.MemorySpace.HBM),
            pl.BlockSpec(memory_space=pltpu.MemorySpace.HBM),
        ],
        out_specs=[
            pl.BlockSpec(memory_space=pltpu.VMEM),
            pl.BlockSpec(memory_space=pltpu.VMEM),
        ],
        out_shape=[
            jax.ShapeDtypeStruct((H, B, D), jnp.float32),
            jax.ShapeDtypeStruct((B, H), jnp.float32),
        ],
        scratch_shapes=[
            pltpu.VMEM((2, np_local, BS, D), jnp.float32),
            pltpu.VMEM((2, np_local, BS, D), jnp.float32),
            pltpu.SemaphoreType.DMA((2,)),
            pltpu.SemaphoreType.DMA((2,)),
        ],
    )(qt, w_tok, K, V)

    out = pl.pallas_call(
        _merge_body,
        out_shape=jax.ShapeDtypeStruct((B, H, D), jnp.float32),
        in_specs=[
            pl.BlockSpec(memory_space=pltpu.VMEM),
            pl.BlockSpec(memory_space=pltpu.VMEM),
        ],
        out_specs=pl.BlockSpec(memory_space=pltpu.VMEM),
        scratch_shapes=[
            pltpu.VMEM((H, B, D), jnp.float32),
            pltpu.VMEM((B, H), jnp.float32),
            pltpu.SemaphoreType.DMA((2,)),
            pltpu.SemaphoreType.DMA((2,)),
        ],
        compiler_params=_CompilerParams(collective_id=0),
    )(acc, l)

    return out[:, None, :, :]


# device time: 52127 ns/iter; 5.3146x vs baseline; 1.0504x over previous
import jax
import jax.numpy as jnp
from jax import lax
from jax.experimental import pallas as pl
from jax.experimental.pallas import tpu as pltpu

B = 32
H = 16
D = 128
BS = 32
X_SIZE = 2

_CompilerParams = getattr(pltpu, "CompilerParams", None) or getattr(
    pltpu, "TPUCompilerParams"
)


def _fused_body(
    qt_ref, w_ref, k_hbm, v_hbm, out_ref,
    acc, l_buf, racc, rl, k_buf, v_buf, k_sems, v_sems, send_sems, recv_sems,
):
    np_local, bs = k_hbm.shape[0], k_hbm.shape[1]
    t = np_local * bs

    my_x = lax.axis_index("x")
    my_y = lax.axis_index("y")
    my_z = lax.axis_index("z")
    partner = (1 - my_x, my_y, my_z)

    def copy(hbm, buf, sems, h, slot):
        return pltpu.make_async_copy(
            hbm.at[:, :, h, :], buf.at[slot], sems.at[slot]
        )

    def send(src, dst, i):
        return pltpu.make_async_remote_copy(
            src_ref=src,
            dst_ref=dst,
            send_sem=send_sems.at[i],
            recv_sem=recv_sems.at[i],
            device_id=partner,
            device_id_type=pl.DeviceIdType.MESH,
        )

    copy(k_hbm, k_buf, k_sems, 0, 0).start()
    copy(v_hbm, v_buf, v_sems, 0, 0).start()

    bar = pltpu.get_barrier_semaphore()
    pl.semaphore_signal(
        bar, inc=1, device_id=partner, device_id_type=pl.DeviceIdType.MESH
    )
    pl.semaphore_wait(bar, 1)

    w = w_ref[...]
    for h in range(H):
        slot = h % 2
        if h + 1 < H:
            copy(k_hbm, k_buf, k_sems, h + 1, 1 - slot).start()
            copy(v_hbm, v_buf, v_sems, h + 1, 1 - slot).start()
        copy(k_hbm, k_buf, k_sems, h, slot).wait()
        q = qt_ref[h]
        k = k_buf[slot].reshape(t, D)
        s = lax.dot_general(
            q, k, (((1,), (1,)), ((), ())),
            preferred_element_type=jnp.float32,
        )
        e = w * jnp.exp(s)
        l_buf[:, h : h + 1] = jnp.sum(e, axis=1, keepdims=True)
        copy(v_hbm, v_buf, v_sems, h, slot).wait()
        v = v_buf[slot].reshape(t, D)
        acc[h] = lax.dot_general(
            e, v, (((1,), (0,)), ((), ())),
            preferred_element_type=jnp.float32,
        )
        send(acc.at[h], racc.at[h], h).start()

    send(l_buf, rl, H).start()

    for h in range(H):
        send(acc.at[h], racc.at[h], h).wait()
    send(l_buf, rl, H).wait()

    l_star = l_buf[...] + rl[...]
    for h in range(H):
        out_ref[:, h, :] = (acc[h] + racc[h]) / l_star[:, h : h + 1]


def kernel(Q, K, V, bt, lens):
    np_local = K.shape[0]
    n_slots = bt.shape[1]
    my_x = lax.axis_index("x")

    base = my_x * np_local
    slot = jnp.arange(n_slots)
    valid = slot[None, :] < lens[:, None]
    local_id = jnp.where(valid, bt, -1) - base
    w = jnp.sum(
        (local_id[:, :, None] == jnp.arange(np_local)[None, None, :]).astype(
            jnp.float32
        ),
        axis=1,
    )
    w_tok = jnp.repeat(w, BS, axis=1)

    qt = Q[:, 0].transpose(1, 0, 2) * (D ** -0.5)

    out = pl.pallas_call(
        _fused_body,
        in_specs=[
            pl.BlockSpec(memory_space=pltpu.VMEM),
            pl.BlockSpec(memory_space=pltpu.VMEM),
            pl.BlockSpec(memory_space=pltpu.MemorySpace.HBM),
            pl.BlockSpec(memory_space=pltpu.MemorySpace.HBM),
        ],
        out_specs=pl.BlockSpec(memory_space=pltpu.VMEM),
        out_shape=jax.ShapeDtypeStruct((B, H, D), jnp.float32),
        scratch_shapes=[
            pltpu.VMEM((H, B, D), jnp.float32),
            pltpu.VMEM((B, H), jnp.float32),
            pltpu.VMEM((H, B, D), jnp.float32),
            pltpu.VMEM((B, H), jnp.float32),
            pltpu.VMEM((2, np_local, BS, D), jnp.float32),
            pltpu.VMEM((2, np_local, BS, D), jnp.float32),
            pltpu.SemaphoreType.DMA((2,)),
            pltpu.SemaphoreType.DMA((2,)),
            pltpu.SemaphoreType.DMA((H + 1,)),
            pltpu.SemaphoreType.DMA((H + 1,)),
        ],
        compiler_params=_CompilerParams(collective_id=0),
    )(qt, w_tok, K, V)

    return out[:, None, :, :]
